# baseline (speedup 1.0000x reference)
import functools

import jax
import jax.numpy as jnp
from jax import lax
from jax.experimental import pallas as pl
from jax.experimental.pallas import tpu as pltpu
from jax.experimental.pallas import tpu_sc as plsc

D_MODEL = 1024
D_FF = 4096
NUM_EXPERTS = 8
TOP_K = 2
T_TOKENS = 4096

M_BLK = 512
R_MAX = TOP_K * T_TOKENS + NUM_EXPERTS * M_BLK
NB = R_MAX // M_BLK
FF_B = 1024
NF = D_FF // FF_B

SC_CORES = 2
SC_SUBCORES = 16
SC_TILES = SC_CORES * SC_SUBCORES
CH = 16


def _router_body(x_ref, wr_ref, br_ref,
                 w0_ref, w1_ref, pos0_ref, pos1_ref, be_ref, nact_ref):
    x = x_ref[...]
    logits = jnp.dot(x, wr_ref[...], preferred_element_type=jnp.float32)
    logits = logits + br_ref[...]
    m = jnp.max(logits, axis=-1, keepdims=True)
    s = jnp.exp(logits - m)
    p = s / jnp.sum(s, axis=-1, keepdims=True)

    T = p.shape[0]
    E = NUM_EXPERTS
    eidx = lax.broadcasted_iota(jnp.int32, (T, E), 1)
    m0 = jnp.max(p, axis=-1, keepdims=True)
    a0 = jnp.min(jnp.where(p == m0, eidx, E), axis=-1, keepdims=True)
    p1 = jnp.where(eidx == a0, -1.0, p)
    m1 = jnp.max(p1, axis=-1, keepdims=True)
    a1 = jnp.min(jnp.where(p1 == m1, eidx, E), axis=-1, keepdims=True)
    denom = m0 + m1
    w0_ref[...] = m0 / denom
    w1_ref[...] = m1 / denom

    mask = jnp.where(eidx == a0, 1.0, 0.0) + jnp.where(eidx == a1, 1.0, 0.0)

    CB = 512
    ri = lax.broadcasted_iota(jnp.int32, (CB, CB), 0)
    ci = lax.broadcasted_iota(jnp.int32, (CB, CB), 1)
    ltri = jnp.where(ci < ri, 1.0, 0.0)
    carry = jnp.zeros((1, E), jnp.float32)
    parts = []
    for bidx in range(T // CB):
        mb = mask[bidx * CB:(bidx + 1) * CB]
        parts.append(jnp.dot(ltri, mb, preferred_element_type=jnp.float32)
                     + carry)
        carry = carry + jnp.sum(mb, axis=0, keepdims=True)
    ranks = jnp.concatenate(parts, axis=0)

    counts = carry
    padded = jnp.floor((counts + (M_BLK - 1)) / M_BLK) * M_BLK

    re8 = lax.broadcasted_iota(jnp.int32, (E, E), 0)
    ce8 = lax.broadcasted_iota(jnp.int32, (E, E), 1)
    off = jnp.sum(jnp.where(ce8 < re8, jnp.broadcast_to(padded, (E, E)), 0.0),
                  axis=1, keepdims=False).reshape(1, E)

    posf = off + ranks
    pos0 = jnp.sum(jnp.where(eidx == a0, posf, 0.0), axis=1, keepdims=True)
    pos1 = jnp.sum(jnp.where(eidx == a1, posf, 0.0), axis=1, keepdims=True)
    pos0_ref[...] = pos0.astype(jnp.int32)
    pos1_ref[...] = pos1.astype(jnp.int32)

    bi = lax.broadcasted_iota(jnp.int32, (NB, E), 0) * M_BLK
    offb = jnp.broadcast_to(off, (NB, E))
    be_ref[...] = (jnp.sum(jnp.where(offb <= bi.astype(jnp.float32), 1, 0),
                           axis=1, keepdims=True).astype(jnp.int32) - 1
                   ).reshape(1, NB)
    total = jnp.sum(padded)
    nact_ref[...] = (total / M_BLK).astype(jnp.int32).reshape(1, 1)


def _router(xt, W_router, b_router):
    T = xt.shape[0]
    outs = pl.pallas_call(
        _router_body,
        out_shape=[
            jax.ShapeDtypeStruct((T, 1), jnp.float32),
            jax.ShapeDtypeStruct((T, 1), jnp.float32),
            jax.ShapeDtypeStruct((T, 1), jnp.int32),
            jax.ShapeDtypeStruct((T, 1), jnp.int32),
            jax.ShapeDtypeStruct((1, NB), jnp.int32),
            jax.ShapeDtypeStruct((1, 1), jnp.int32),
        ],
    )(xt, W_router, b_router.reshape(1, NUM_EXPERTS))
    return outs


def _dispatch_sc(xt, pos0, pos1):
    mesh = plsc.VectorSubcoreMesh(core_axis_name="c", subcore_axis_name="s")
    per_w = T_TOKENS // SC_TILES

    @functools.partial(
        pl.kernel,
        out_type=jax.ShapeDtypeStruct((R_MAX, D_MODEL), jnp.float32),
        mesh=mesh,
        scratch_types=[
            pltpu.VMEM((CH,), jnp.int32),
            pltpu.VMEM((CH,), jnp.int32),
            pltpu.VMEM((CH, D_MODEL), jnp.float32),
        ],
    )
    def scatter_kernel(x_hbm, p0_hbm, p1_hbm, out_hbm, i0_v, i1_v, rows_v):
        wid = lax.axis_index("s") * SC_CORES + lax.axis_index("c")
        base0 = wid * per_w

        @pl.loop(0, per_w, step=CH)
        def _(it):
            base = base0 + it
            pltpu.sync_copy(p0_hbm.at[pl.ds(base, CH)], i0_v)
            pltpu.sync_copy(p1_hbm.at[pl.ds(base, CH)], i1_v)
            pltpu.sync_copy(x_hbm.at[pl.ds(base, CH)], rows_v)
            pltpu.sync_copy(rows_v, out_hbm.at[i0_v])
            pltpu.sync_copy(rows_v, out_hbm.at[i1_v])

    return scatter_kernel(xt, pos0.reshape(-1), pos1.reshape(-1))


def _gemm_body(be_ref, nact_ref, xs_ref, wg_ref, bg_ref, wu_ref, bu_ref,
               wd_ref, bd_ref, out_ref):
    i = pl.program_id(0)
    j = pl.program_id(1)

    @pl.when(i < nact_ref[0])
    def _():
        @pl.when(j == 0)
        def _():
            out_ref[...] = jnp.zeros_like(out_ref)

        x = xs_ref[...].astype(jnp.bfloat16)
        hg = jnp.dot(x, wg_ref[0], preferred_element_type=jnp.float32)
        hg = hg + bg_ref[0]
        hu = jnp.dot(x, wu_ref[0], preferred_element_type=jnp.float32)
        hu = hu + bu_ref[0]
        h = (hg * jax.nn.sigmoid(hg)) * hu
        out_ref[...] += jnp.dot(h.astype(jnp.bfloat16), wd_ref[0],
                                preferred_element_type=jnp.float32)

        @pl.when(j == 0)
        def _():
            out_ref[...] += bd_ref[0]


def _grouped_gemm(xs, be, nact, W_gate, b_gate, W_up, b_up, W_down, b_down):
    d = D_MODEL

    def xmap(i, j, be_r, na_r):
        return (jnp.minimum(i, na_r[0] - 1), 0)

    def wmap_g(i, j, be_r, na_r):
        return (be_r[i], 0, j)

    def bmap_g(i, j, be_r, na_r):
        return (be_r[i], 0, j)

    def wmap_d(i, j, be_r, na_r):
        return (be_r[i], j, 0)

    def bmap_d(i, j, be_r, na_r):
        return (be_r[i], 0, 0)

    grid_spec = pltpu.PrefetchScalarGridSpec(
        num_scalar_prefetch=2,
        grid=(NB, NF),
        in_specs=[
            pl.BlockSpec((M_BLK, d), xmap),
            pl.BlockSpec((1, d, FF_B), wmap_g),
            pl.BlockSpec((1, 1, FF_B), bmap_g),
            pl.BlockSpec((1, d, FF_B), wmap_g),
            pl.BlockSpec((1, 1, FF_B), bmap_g),
            pl.BlockSpec((1, FF_B, d), wmap_d),
            pl.BlockSpec((1, 1, d), bmap_d),
        ],
        out_specs=pl.BlockSpec((M_BLK, d), lambda i, j, be_r, na_r: (i, 0)),
    )
    return pl.pallas_call(
        _gemm_body,
        grid_spec=grid_spec,
        out_shape=jax.ShapeDtypeStruct((R_MAX, d), jnp.float32),
    )(be.reshape(NB), nact.reshape(1), xs, W_gate.astype(jnp.bfloat16),
      b_gate.reshape(NUM_EXPERTS, 1, D_FF), W_up.astype(jnp.bfloat16),
      b_up.reshape(NUM_EXPERTS, 1, D_FF), W_down.astype(jnp.bfloat16),
      b_down.reshape(NUM_EXPERTS, 1, D_MODEL))


def _combine_gather_sc(o_sorted, pos0, pos1):
    mesh = plsc.VectorSubcoreMesh(core_axis_name="c", subcore_axis_name="s")
    per_w = T_TOKENS // SC_TILES

    @functools.partial(
        pl.kernel,
        out_type=[
            jax.ShapeDtypeStruct((T_TOKENS, D_MODEL), jnp.float32),
            jax.ShapeDtypeStruct((T_TOKENS, D_MODEL), jnp.float32),
        ],
        mesh=mesh,
        scratch_types=[
            pltpu.VMEM((CH,), jnp.int32),
            pltpu.VMEM((CH,), jnp.int32),
            pltpu.VMEM((CH, D_MODEL), jnp.float32),
            pltpu.VMEM((CH, D_MODEL), jnp.float32),
        ],
    )
    def gather_kernel(os_hbm, p0_hbm, p1_hbm, g0_hbm, g1_hbm,
                      i0_v, i1_v, r0_v, r1_v):
        wid = lax.axis_index("s") * SC_CORES + lax.axis_index("c")
        base0 = wid * per_w

        @pl.loop(0, per_w, step=CH)
        def _(it):
            base = base0 + it
            pltpu.sync_copy(p0_hbm.at[pl.ds(base, CH)], i0_v)
            pltpu.sync_copy(p1_hbm.at[pl.ds(base, CH)], i1_v)
            pltpu.sync_copy(os_hbm.at[i0_v], r0_v)
            pltpu.sync_copy(os_hbm.at[i1_v], r1_v)
            pltpu.sync_copy(r0_v, g0_hbm.at[pl.ds(base, CH)])
            pltpu.sync_copy(r1_v, g1_hbm.at[pl.ds(base, CH)])

    return gather_kernel(o_sorted, pos0.reshape(-1), pos1.reshape(-1))


def _combine_body(g0_ref, g1_ref, w0_ref, w1_ref, out_ref):
    out_ref[...] = w0_ref[...] * g0_ref[...] + w1_ref[...] * g1_ref[...]


def _combine(g0, g1, w0, w1):
    TB = 2048
    return pl.pallas_call(
        _combine_body,
        grid=(T_TOKENS // TB,),
        in_specs=[
            pl.BlockSpec((TB, D_MODEL), lambda t: (t, 0)),
            pl.BlockSpec((TB, D_MODEL), lambda t: (t, 0)),
            pl.BlockSpec((TB, 1), lambda t: (t, 0)),
            pl.BlockSpec((TB, 1), lambda t: (t, 0)),
        ],
        out_specs=pl.BlockSpec((TB, D_MODEL), lambda t: (t, 0)),
        out_shape=jax.ShapeDtypeStruct((T_TOKENS, D_MODEL), jnp.float32),
    )(g0, g1, w0, w1)


def kernel(x, W_router, b_router, W_gate, b_gate, W_up, b_up, W_down, b_down):
    b, s, d = x.shape
    xt = x.reshape(-1, d)
    w0, w1, pos0, pos1, be, nact = _router(xt, W_router, b_router)
    xs = _dispatch_sc(xt, pos0, pos1)
    o_sorted = _grouped_gemm(xs, be, nact, W_gate, b_gate, W_up, b_up,
                             W_down, b_down)
    g0, g1 = _combine_gather_sc(o_sorted, pos0, pos1)
    out = _combine(g0, g1, w0, w1)
    return out.reshape(b, s, d)


# speedup vs baseline: 1.1623x; 1.1623x over previous
import functools

import jax
import jax.numpy as jnp
from jax import lax
from jax.experimental import pallas as pl
from jax.experimental.pallas import tpu as pltpu
from jax.experimental.pallas import tpu_sc as plsc

D_MODEL = 1024
D_FF = 4096
NUM_EXPERTS = 8
TOP_K = 2
T_TOKENS = 4096

M_BLK = 512
R_MAX = TOP_K * T_TOKENS + NUM_EXPERTS * M_BLK
NB = R_MAX // M_BLK
FF_B = 1024
NF = D_FF // FF_B

SC_CORES = 2
SC_SUBCORES = 16
SC_TILES = SC_CORES * SC_SUBCORES
CH = 16


def _router_body(x_ref, wr_ref, br_ref,
                 w0_ref, w1_ref, pos0_ref, pos1_ref, be_ref, nact_ref):
    x = x_ref[...]
    logits = jnp.dot(x, wr_ref[...], preferred_element_type=jnp.float32)
    logits = logits + br_ref[...]
    m = jnp.max(logits, axis=-1, keepdims=True)
    s = jnp.exp(logits - m)
    p = s / jnp.sum(s, axis=-1, keepdims=True)

    T = p.shape[0]
    E = NUM_EXPERTS
    eidx = lax.broadcasted_iota(jnp.int32, (T, E), 1)
    m0 = jnp.max(p, axis=-1, keepdims=True)
    a0 = jnp.min(jnp.where(p == m0, eidx, E), axis=-1, keepdims=True)
    p1 = jnp.where(eidx == a0, -1.0, p)
    m1 = jnp.max(p1, axis=-1, keepdims=True)
    a1 = jnp.min(jnp.where(p1 == m1, eidx, E), axis=-1, keepdims=True)
    denom = m0 + m1
    w0_ref[...] = m0 / denom
    w1_ref[...] = m1 / denom

    mask = jnp.where(eidx == a0, 1.0, 0.0) + jnp.where(eidx == a1, 1.0, 0.0)

    CB = 512
    ri = lax.broadcasted_iota(jnp.int32, (CB, CB), 0)
    ci = lax.broadcasted_iota(jnp.int32, (CB, CB), 1)
    ltri = jnp.where(ci < ri, 1.0, 0.0)
    carry = jnp.zeros((1, E), jnp.float32)
    parts = []
    for bidx in range(T // CB):
        mb = mask[bidx * CB:(bidx + 1) * CB]
        parts.append(jnp.dot(ltri, mb, preferred_element_type=jnp.float32)
                     + carry)
        carry = carry + jnp.sum(mb, axis=0, keepdims=True)
    ranks = jnp.concatenate(parts, axis=0)

    counts = carry
    padded = jnp.floor((counts + (M_BLK - 1)) / M_BLK) * M_BLK

    re8 = lax.broadcasted_iota(jnp.int32, (E, E), 0)
    ce8 = lax.broadcasted_iota(jnp.int32, (E, E), 1)
    off = jnp.sum(jnp.where(ce8 < re8, jnp.broadcast_to(padded, (E, E)), 0.0),
                  axis=1, keepdims=False).reshape(1, E)

    posf = off + ranks
    pos0 = jnp.sum(jnp.where(eidx == a0, posf, 0.0), axis=1, keepdims=True)
    pos1 = jnp.sum(jnp.where(eidx == a1, posf, 0.0), axis=1, keepdims=True)
    pos0_ref[...] = pos0.astype(jnp.int32)
    pos1_ref[...] = pos1.astype(jnp.int32)

    bi = lax.broadcasted_iota(jnp.int32, (NB, E), 0) * M_BLK
    offb = jnp.broadcast_to(off, (NB, E))
    be_ref[...] = (jnp.sum(jnp.where(offb <= bi.astype(jnp.float32), 1, 0),
                           axis=1, keepdims=True).astype(jnp.int32) - 1
                   ).reshape(1, NB)
    total = jnp.sum(padded)
    nact_ref[...] = (total / M_BLK).astype(jnp.int32).reshape(1, 1)


def _router(xt, W_router, b_router):
    T = xt.shape[0]
    outs = pl.pallas_call(
        _router_body,
        out_shape=[
            jax.ShapeDtypeStruct((T, 1), jnp.float32),
            jax.ShapeDtypeStruct((T, 1), jnp.float32),
            jax.ShapeDtypeStruct((T, 1), jnp.int32),
            jax.ShapeDtypeStruct((T, 1), jnp.int32),
            jax.ShapeDtypeStruct((1, NB), jnp.int32),
            jax.ShapeDtypeStruct((1, 1), jnp.int32),
        ],
    )(xt, W_router, b_router.reshape(1, NUM_EXPERTS))
    return outs


def _dispatch_sc(xt, pos0, pos1):
    mesh = plsc.VectorSubcoreMesh(core_axis_name="c", subcore_axis_name="s")
    per_w = T_TOKENS // SC_TILES

    @functools.partial(
        pl.kernel,
        out_type=jax.ShapeDtypeStruct((R_MAX, D_MODEL), jnp.float32),
        mesh=mesh,
        scratch_types=[
            pltpu.VMEM((CH,), jnp.int32),
            pltpu.VMEM((CH,), jnp.int32),
            pltpu.VMEM((CH, D_MODEL), jnp.float32),
        ],
    )
    def scatter_kernel(x_hbm, p0_hbm, p1_hbm, out_hbm, i0_v, i1_v, rows_v):
        wid = lax.axis_index("s") * SC_CORES + lax.axis_index("c")
        base0 = wid * per_w

        @pl.loop(0, per_w, step=CH)
        def _(it):
            base = base0 + it
            pltpu.sync_copy(p0_hbm.at[pl.ds(base, CH)], i0_v)
            pltpu.sync_copy(p1_hbm.at[pl.ds(base, CH)], i1_v)
            pltpu.sync_copy(x_hbm.at[pl.ds(base, CH)], rows_v)
            pltpu.sync_copy(rows_v, out_hbm.at[i0_v])
            pltpu.sync_copy(rows_v, out_hbm.at[i1_v])

    return scatter_kernel(xt, pos0.reshape(-1), pos1.reshape(-1))


def _gemm_body(be_ref, nact_ref, xs_ref, wg_ref, bg_ref, wu_ref, bu_ref,
               wd_ref, bd_ref, out_ref):
    i = pl.program_id(0)
    j = pl.program_id(1)

    @pl.when(i < nact_ref[0])
    def _():
        @pl.when(j == 0)
        def _():
            out_ref[...] = jnp.zeros_like(out_ref)

        x = xs_ref[...].astype(jnp.bfloat16)
        hg = jnp.dot(x, wg_ref[0].astype(jnp.bfloat16),
                     preferred_element_type=jnp.float32)
        hg = hg + bg_ref[0]
        hu = jnp.dot(x, wu_ref[0].astype(jnp.bfloat16),
                     preferred_element_type=jnp.float32)
        hu = hu + bu_ref[0]
        h = (hg * jax.nn.sigmoid(hg)) * hu
        out_ref[...] += jnp.dot(h.astype(jnp.bfloat16),
                                wd_ref[0].astype(jnp.bfloat16),
                                preferred_element_type=jnp.float32)

        @pl.when(j == 0)
        def _():
            out_ref[...] += bd_ref[0]


def _grouped_gemm(xs, be, nact, W_gate, b_gate, W_up, b_up, W_down, b_down):
    d = D_MODEL

    def xmap(i, j, be_r, na_r):
        return (jnp.minimum(i, na_r[0] - 1), 0)

    def wmap_g(i, j, be_r, na_r):
        return (be_r[i], 0, j)

    def bmap_g(i, j, be_r, na_r):
        return (be_r[i], 0, j)

    def wmap_d(i, j, be_r, na_r):
        return (be_r[i], j, 0)

    def bmap_d(i, j, be_r, na_r):
        return (be_r[i], 0, 0)

    grid_spec = pltpu.PrefetchScalarGridSpec(
        num_scalar_prefetch=2,
        grid=(NB, NF),
        in_specs=[
            pl.BlockSpec((M_BLK, d), xmap),
            pl.BlockSpec((1, d, FF_B), wmap_g),
            pl.BlockSpec((1, 1, FF_B), bmap_g),
            pl.BlockSpec((1, d, FF_B), wmap_g),
            pl.BlockSpec((1, 1, FF_B), bmap_g),
            pl.BlockSpec((1, FF_B, d), wmap_d),
            pl.BlockSpec((1, 1, d), bmap_d),
        ],
        out_specs=pl.BlockSpec((M_BLK, d), lambda i, j, be_r, na_r: (i, 0)),
    )
    return pl.pallas_call(
        _gemm_body,
        grid_spec=grid_spec,
        out_shape=jax.ShapeDtypeStruct((R_MAX, d), jnp.float32),
    )(be.reshape(NB), nact.reshape(1), xs, W_gate,
      b_gate.reshape(NUM_EXPERTS, 1, D_FF), W_up,
      b_up.reshape(NUM_EXPERTS, 1, D_FF), W_down,
      b_down.reshape(NUM_EXPERTS, 1, D_MODEL))


def _combine_gather_sc(o_sorted, pos0, pos1):
    mesh = plsc.VectorSubcoreMesh(core_axis_name="c", subcore_axis_name="s")
    per_w = T_TOKENS // SC_TILES

    @functools.partial(
        pl.kernel,
        out_type=[
            jax.ShapeDtypeStruct((T_TOKENS, D_MODEL), jnp.float32),
            jax.ShapeDtypeStruct((T_TOKENS, D_MODEL), jnp.float32),
        ],
        mesh=mesh,
        scratch_types=[
            pltpu.VMEM((CH,), jnp.int32),
            pltpu.VMEM((CH,), jnp.int32),
            pltpu.VMEM((CH, D_MODEL), jnp.float32),
            pltpu.VMEM((CH, D_MODEL), jnp.float32),
        ],
    )
    def gather_kernel(os_hbm, p0_hbm, p1_hbm, g0_hbm, g1_hbm,
                      i0_v, i1_v, r0_v, r1_v):
        wid = lax.axis_index("s") * SC_CORES + lax.axis_index("c")
        base0 = wid * per_w

        @pl.loop(0, per_w, step=CH)
        def _(it):
            base = base0 + it
            pltpu.sync_copy(p0_hbm.at[pl.ds(base, CH)], i0_v)
            pltpu.sync_copy(p1_hbm.at[pl.ds(base, CH)], i1_v)
            pltpu.sync_copy(os_hbm.at[i0_v], r0_v)
            pltpu.sync_copy(os_hbm.at[i1_v], r1_v)
            pltpu.sync_copy(r0_v, g0_hbm.at[pl.ds(base, CH)])
            pltpu.sync_copy(r1_v, g1_hbm.at[pl.ds(base, CH)])

    return gather_kernel(o_sorted, pos0.reshape(-1), pos1.reshape(-1))


def _combine_body(g0_ref, g1_ref, w0_ref, w1_ref, out_ref):
    out_ref[...] = w0_ref[...] * g0_ref[...] + w1_ref[...] * g1_ref[...]


def _combine(g0, g1, w0, w1):
    TB = 2048
    return pl.pallas_call(
        _combine_body,
        grid=(T_TOKENS // TB,),
        in_specs=[
            pl.BlockSpec((TB, D_MODEL), lambda t: (t, 0)),
            pl.BlockSpec((TB, D_MODEL), lambda t: (t, 0)),
            pl.BlockSpec((TB, 1), lambda t: (t, 0)),
            pl.BlockSpec((TB, 1), lambda t: (t, 0)),
        ],
        out_specs=pl.BlockSpec((TB, D_MODEL), lambda t: (t, 0)),
        out_shape=jax.ShapeDtypeStruct((T_TOKENS, D_MODEL), jnp.float32),
    )(g0, g1, w0, w1)


def kernel(x, W_router, b_router, W_gate, b_gate, W_up, b_up, W_down, b_down):
    b, s, d = x.shape
    xt = x.reshape(-1, d)
    w0, w1, pos0, pos1, be, nact = _router(xt, W_router, b_router)
    xs = _dispatch_sc(xt, pos0, pos1)
    o_sorted = _grouped_gemm(xs, be, nact, W_gate, b_gate, W_up, b_up,
                             W_down, b_down)
    g0, g1 = _combine_gather_sc(o_sorted, pos0, pos1)
    out = _combine(g0, g1, w0, w1)
    return out.reshape(b, s, d)


# speedup vs baseline: 1.1801x; 1.0153x over previous
import functools

import jax
import jax.numpy as jnp
from jax import lax
from jax.experimental import pallas as pl
from jax.experimental.pallas import tpu as pltpu
from jax.experimental.pallas import tpu_sc as plsc

D_MODEL = 1024
D_FF = 4096
NUM_EXPERTS = 8
TOP_K = 2
T_TOKENS = 4096

M_BLK = 1024
R_MAX = TOP_K * T_TOKENS + NUM_EXPERTS * M_BLK
NB = R_MAX // M_BLK
FF_B = 1024
NF = D_FF // FF_B

SC_CORES = 2
SC_SUBCORES = 16
SC_TILES = SC_CORES * SC_SUBCORES
CH = 16


def _router_body(x_ref, wr_ref, br_ref,
                 w0_ref, w1_ref, pos0_ref, pos1_ref, be_ref, nact_ref):
    x = x_ref[...]
    logits = jnp.dot(x, wr_ref[...], preferred_element_type=jnp.float32)
    logits = logits + br_ref[...]
    m = jnp.max(logits, axis=-1, keepdims=True)
    s = jnp.exp(logits - m)
    p = s / jnp.sum(s, axis=-1, keepdims=True)

    T = p.shape[0]
    E = NUM_EXPERTS
    eidx = lax.broadcasted_iota(jnp.int32, (T, E), 1)
    m0 = jnp.max(p, axis=-1, keepdims=True)
    a0 = jnp.min(jnp.where(p == m0, eidx, E), axis=-1, keepdims=True)
    p1 = jnp.where(eidx == a0, -1.0, p)
    m1 = jnp.max(p1, axis=-1, keepdims=True)
    a1 = jnp.min(jnp.where(p1 == m1, eidx, E), axis=-1, keepdims=True)
    denom = m0 + m1
    w0_ref[...] = m0 / denom
    w1_ref[...] = m1 / denom

    mask = jnp.where(eidx == a0, 1.0, 0.0) + jnp.where(eidx == a1, 1.0, 0.0)

    CB = 512
    ri = lax.broadcasted_iota(jnp.int32, (CB, CB), 0)
    ci = lax.broadcasted_iota(jnp.int32, (CB, CB), 1)
    ltri = jnp.where(ci < ri, 1.0, 0.0)
    carry = jnp.zeros((1, E), jnp.float32)
    parts = []
    for bidx in range(T // CB):
        mb = mask[bidx * CB:(bidx + 1) * CB]
        parts.append(jnp.dot(ltri, mb, preferred_element_type=jnp.float32)
                     + carry)
        carry = carry + jnp.sum(mb, axis=0, keepdims=True)
    ranks = jnp.concatenate(parts, axis=0)

    counts = carry
    padded = jnp.floor((counts + (M_BLK - 1)) / M_BLK) * M_BLK

    re8 = lax.broadcasted_iota(jnp.int32, (E, E), 0)
    ce8 = lax.broadcasted_iota(jnp.int32, (E, E), 1)
    off = jnp.sum(jnp.where(ce8 < re8, jnp.broadcast_to(padded, (E, E)), 0.0),
                  axis=1, keepdims=False).reshape(1, E)

    posf = off + ranks
    pos0 = jnp.sum(jnp.where(eidx == a0, posf, 0.0), axis=1, keepdims=True)
    pos1 = jnp.sum(jnp.where(eidx == a1, posf, 0.0), axis=1, keepdims=True)
    pos0_ref[...] = pos0.astype(jnp.int32)
    pos1_ref[...] = pos1.astype(jnp.int32)

    bi = lax.broadcasted_iota(jnp.int32, (NB, E), 0) * M_BLK
    offb = jnp.broadcast_to(off, (NB, E))
    be_ref[...] = (jnp.sum(jnp.where(offb <= bi.astype(jnp.float32), 1, 0),
                           axis=1, keepdims=True).astype(jnp.int32) - 1
                   ).reshape(1, NB)
    total = jnp.sum(padded)
    nact_ref[...] = (total / M_BLK).astype(jnp.int32).reshape(1, 1)


def _router(xt, W_router, b_router):
    T = xt.shape[0]
    outs = pl.pallas_call(
        _router_body,
        out_shape=[
            jax.ShapeDtypeStruct((T, 1), jnp.float32),
            jax.ShapeDtypeStruct((T, 1), jnp.float32),
            jax.ShapeDtypeStruct((T, 1), jnp.int32),
            jax.ShapeDtypeStruct((T, 1), jnp.int32),
            jax.ShapeDtypeStruct((1, NB), jnp.int32),
            jax.ShapeDtypeStruct((1, 1), jnp.int32),
        ],
    )(xt, W_router, b_router.reshape(1, NUM_EXPERTS))
    return outs


def _dispatch_sc(xt, pos0, pos1):
    mesh = plsc.VectorSubcoreMesh(core_axis_name="c", subcore_axis_name="s")
    per_w = T_TOKENS // SC_TILES

    @functools.partial(
        pl.kernel,
        out_type=jax.ShapeDtypeStruct((R_MAX, D_MODEL), jnp.float32),
        mesh=mesh,
        scratch_types=[
            pltpu.VMEM((CH,), jnp.int32),
            pltpu.VMEM((CH,), jnp.int32),
            pltpu.VMEM((CH, D_MODEL), jnp.float32),
        ],
    )
    def scatter_kernel(x_hbm, p0_hbm, p1_hbm, out_hbm, i0_v, i1_v, rows_v):
        wid = lax.axis_index("s") * SC_CORES + lax.axis_index("c")
        base0 = wid * per_w

        @pl.loop(0, per_w, step=CH)
        def _(it):
            base = base0 + it
            pltpu.sync_copy(p0_hbm.at[pl.ds(base, CH)], i0_v)
            pltpu.sync_copy(p1_hbm.at[pl.ds(base, CH)], i1_v)
            pltpu.sync_copy(x_hbm.at[pl.ds(base, CH)], rows_v)
            pltpu.sync_copy(rows_v, out_hbm.at[i0_v])
            pltpu.sync_copy(rows_v, out_hbm.at[i1_v])

    return scatter_kernel(xt, pos0.reshape(-1), pos1.reshape(-1))


def _gemm_body(be_ref, nact_ref, xs_ref, wg_ref, bg_ref, wu_ref, bu_ref,
               wd_ref, bd_ref, out_ref):
    i = pl.program_id(0)
    j = pl.program_id(1)

    @pl.when(i < nact_ref[0])
    def _():
        @pl.when(j == 0)
        def _():
            out_ref[...] = jnp.zeros_like(out_ref)

        x = xs_ref[...].astype(jnp.bfloat16)
        hg = jnp.dot(x, wg_ref[0].astype(jnp.bfloat16),
                     preferred_element_type=jnp.float32)
        hg = hg + bg_ref[0]
        hu = jnp.dot(x, wu_ref[0].astype(jnp.bfloat16),
                     preferred_element_type=jnp.float32)
        hu = hu + bu_ref[0]
        h = (hg * jax.nn.sigmoid(hg)) * hu
        out_ref[...] += jnp.dot(h.astype(jnp.bfloat16),
                                wd_ref[0].astype(jnp.bfloat16),
                                preferred_element_type=jnp.float32)

        @pl.when(j == 0)
        def _():
            out_ref[...] += bd_ref[0]


def _grouped_gemm(xs, be, nact, W_gate, b_gate, W_up, b_up, W_down, b_down):
    d = D_MODEL

    def xmap(i, j, be_r, na_r):
        return (jnp.minimum(i, na_r[0] - 1), 0)

    def wmap_g(i, j, be_r, na_r):
        return (be_r[i], 0, j)

    def bmap_g(i, j, be_r, na_r):
        return (be_r[i], 0, j)

    def wmap_d(i, j, be_r, na_r):
        return (be_r[i], j, 0)

    def bmap_d(i, j, be_r, na_r):
        return (be_r[i], 0, 0)

    grid_spec = pltpu.PrefetchScalarGridSpec(
        num_scalar_prefetch=2,
        grid=(NB, NF),
        in_specs=[
            pl.BlockSpec((M_BLK, d), xmap),
            pl.BlockSpec((1, d, FF_B), wmap_g),
            pl.BlockSpec((1, 1, FF_B), bmap_g),
            pl.BlockSpec((1, d, FF_B), wmap_g),
            pl.BlockSpec((1, 1, FF_B), bmap_g),
            pl.BlockSpec((1, FF_B, d), wmap_d),
            pl.BlockSpec((1, 1, d), bmap_d),
        ],
        out_specs=pl.BlockSpec((M_BLK, d), lambda i, j, be_r, na_r: (i, 0)),
    )
    return pl.pallas_call(
        _gemm_body,
        grid_spec=grid_spec,
        out_shape=jax.ShapeDtypeStruct((R_MAX, d), jnp.float32),
    )(be.reshape(NB), nact.reshape(1), xs, W_gate,
      b_gate.reshape(NUM_EXPERTS, 1, D_FF), W_up,
      b_up.reshape(NUM_EXPERTS, 1, D_FF), W_down,
      b_down.reshape(NUM_EXPERTS, 1, D_MODEL))


def _combine_gather_sc(o_sorted, pos0, pos1):
    mesh = plsc.VectorSubcoreMesh(core_axis_name="c", subcore_axis_name="s")
    per_w = T_TOKENS // SC_TILES

    @functools.partial(
        pl.kernel,
        out_type=[
            jax.ShapeDtypeStruct((T_TOKENS, D_MODEL), jnp.float32),
            jax.ShapeDtypeStruct((T_TOKENS, D_MODEL), jnp.float32),
        ],
        mesh=mesh,
        scratch_types=[
            pltpu.VMEM((CH,), jnp.int32),
            pltpu.VMEM((CH,), jnp.int32),
            pltpu.VMEM((CH, D_MODEL), jnp.float32),
            pltpu.VMEM((CH, D_MODEL), jnp.float32),
        ],
    )
    def gather_kernel(os_hbm, p0_hbm, p1_hbm, g0_hbm, g1_hbm,
                      i0_v, i1_v, r0_v, r1_v):
        wid = lax.axis_index("s") * SC_CORES + lax.axis_index("c")
        base0 = wid * per_w

        @pl.loop(0, per_w, step=CH)
        def _(it):
            base = base0 + it
            pltpu.sync_copy(p0_hbm.at[pl.ds(base, CH)], i0_v)
            pltpu.sync_copy(p1_hbm.at[pl.ds(base, CH)], i1_v)
            pltpu.sync_copy(os_hbm.at[i0_v], r0_v)
            pltpu.sync_copy(os_hbm.at[i1_v], r1_v)
            pltpu.sync_copy(r0_v, g0_hbm.at[pl.ds(base, CH)])
            pltpu.sync_copy(r1_v, g1_hbm.at[pl.ds(base, CH)])

    return gather_kernel(o_sorted, pos0.reshape(-1), pos1.reshape(-1))


def _combine_body(g0_ref, g1_ref, w0_ref, w1_ref, out_ref):
    out_ref[...] = w0_ref[...] * g0_ref[...] + w1_ref[...] * g1_ref[...]


def _combine(g0, g1, w0, w1):
    TB = 2048
    return pl.pallas_call(
        _combine_body,
        grid=(T_TOKENS // TB,),
        in_specs=[
            pl.BlockSpec((TB, D_MODEL), lambda t: (t, 0)),
            pl.BlockSpec((TB, D_MODEL), lambda t: (t, 0)),
            pl.BlockSpec((TB, 1), lambda t: (t, 0)),
            pl.BlockSpec((TB, 1), lambda t: (t, 0)),
        ],
        out_specs=pl.BlockSpec((TB, D_MODEL), lambda t: (t, 0)),
        out_shape=jax.ShapeDtypeStruct((T_TOKENS, D_MODEL), jnp.float32),
    )(g0, g1, w0, w1)


def kernel(x, W_router, b_router, W_gate, b_gate, W_up, b_up, W_down, b_down):
    b, s, d = x.shape
    xt = x.reshape(-1, d)
    w0, w1, pos0, pos1, be, nact = _router(xt, W_router, b_router)
    xs = _dispatch_sc(xt, pos0, pos1)
    o_sorted = _grouped_gemm(xs, be, nact, W_gate, b_gate, W_up, b_up,
                             W_down, b_down)
    g0, g1 = _combine_gather_sc(o_sorted, pos0, pos1)
    out = _combine(g0, g1, w0, w1)
    return out.reshape(b, s, d)


# speedup vs baseline: 1.3035x; 1.1046x over previous
import functools

import jax
import jax.numpy as jnp
from jax import lax
from jax.experimental import pallas as pl
from jax.experimental.pallas import tpu as pltpu
from jax.experimental.pallas import tpu_sc as plsc

D_MODEL = 1024
D_FF = 4096
NUM_EXPERTS = 8
TOP_K = 2
T_TOKENS = 4096

M_BLK = 1024
R_MAX = TOP_K * T_TOKENS + NUM_EXPERTS * M_BLK
NB = R_MAX // M_BLK
FF_B = 1024
NF = D_FF // FF_B

SC_CORES = 2
SC_SUBCORES = 16
SC_TILES = SC_CORES * SC_SUBCORES
CH = 16


def _router_body(x_ref, wr_ref, br_ref,
                 w0_ref, w1_ref, pos0_ref, pos1_ref, be_ref, nact_ref):
    x = x_ref[...]
    logits = jnp.dot(x, wr_ref[...], preferred_element_type=jnp.float32)
    logits = logits + br_ref[...]
    m = jnp.max(logits, axis=-1, keepdims=True)
    s = jnp.exp(logits - m)
    p = s / jnp.sum(s, axis=-1, keepdims=True)

    T = p.shape[0]
    E = NUM_EXPERTS
    eidx = lax.broadcasted_iota(jnp.int32, (T, E), 1)
    m0 = jnp.max(p, axis=-1, keepdims=True)
    a0 = jnp.min(jnp.where(p == m0, eidx, E), axis=-1, keepdims=True)
    p1 = jnp.where(eidx == a0, -1.0, p)
    m1 = jnp.max(p1, axis=-1, keepdims=True)
    a1 = jnp.min(jnp.where(p1 == m1, eidx, E), axis=-1, keepdims=True)
    denom = m0 + m1
    w0_ref[...] = m0 / denom
    w1_ref[...] = m1 / denom

    mask = jnp.where(eidx == a0, 1.0, 0.0) + jnp.where(eidx == a1, 1.0, 0.0)

    CB = 512
    ri = lax.broadcasted_iota(jnp.int32, (CB, CB), 0)
    ci = lax.broadcasted_iota(jnp.int32, (CB, CB), 1)
    ltri = jnp.where(ci < ri, 1.0, 0.0)
    carry = jnp.zeros((1, E), jnp.float32)
    parts = []
    for bidx in range(T // CB):
        mb = mask[bidx * CB:(bidx + 1) * CB]
        parts.append(jnp.dot(ltri, mb, preferred_element_type=jnp.float32)
                     + carry)
        carry = carry + jnp.sum(mb, axis=0, keepdims=True)
    ranks = jnp.concatenate(parts, axis=0)

    counts = carry
    padded = jnp.floor((counts + (M_BLK - 1)) / M_BLK) * M_BLK

    re8 = lax.broadcasted_iota(jnp.int32, (E, E), 0)
    ce8 = lax.broadcasted_iota(jnp.int32, (E, E), 1)
    off = jnp.sum(jnp.where(ce8 < re8, jnp.broadcast_to(padded, (E, E)), 0.0),
                  axis=1, keepdims=False).reshape(1, E)

    posf = off + ranks
    pos0 = jnp.sum(jnp.where(eidx == a0, posf, 0.0), axis=1, keepdims=True)
    pos1 = jnp.sum(jnp.where(eidx == a1, posf, 0.0), axis=1, keepdims=True)
    pos0_ref[...] = pos0.astype(jnp.int32)
    pos1_ref[...] = pos1.astype(jnp.int32)

    bi = lax.broadcasted_iota(jnp.int32, (NB, E), 0) * M_BLK
    offb = jnp.broadcast_to(off, (NB, E))
    be_ref[...] = (jnp.sum(jnp.where(offb <= bi.astype(jnp.float32), 1, 0),
                           axis=1, keepdims=True).astype(jnp.int32) - 1
                   ).reshape(1, NB)
    total = jnp.sum(padded)
    nact_ref[...] = (total / M_BLK).astype(jnp.int32).reshape(1, 1)


def _router(xt, W_router, b_router):
    T = xt.shape[0]
    outs = pl.pallas_call(
        _router_body,
        out_shape=[
            jax.ShapeDtypeStruct((T, 1), jnp.float32),
            jax.ShapeDtypeStruct((T, 1), jnp.float32),
            jax.ShapeDtypeStruct((T, 1), jnp.int32),
            jax.ShapeDtypeStruct((T, 1), jnp.int32),
            jax.ShapeDtypeStruct((1, NB), jnp.int32),
            jax.ShapeDtypeStruct((1, 1), jnp.int32),
        ],
    )(xt, W_router, b_router.reshape(1, NUM_EXPERTS))
    return outs


def _dispatch_sc(xt, pos0, pos1):
    mesh = plsc.VectorSubcoreMesh(core_axis_name="c", subcore_axis_name="s")
    per_w = T_TOKENS // SC_TILES

    @functools.partial(
        pl.kernel,
        out_type=jax.ShapeDtypeStruct((R_MAX, D_MODEL), jnp.float32),
        mesh=mesh,
        scratch_types=[
            pltpu.VMEM((CH,), jnp.int32),
            pltpu.VMEM((CH,), jnp.int32),
            pltpu.VMEM((CH, D_MODEL), jnp.float32),
        ],
    )
    def scatter_kernel(x_hbm, p0_hbm, p1_hbm, out_hbm, i0_v, i1_v, rows_v):
        wid = lax.axis_index("s") * SC_CORES + lax.axis_index("c")
        base0 = wid * per_w

        @pl.loop(0, per_w, step=CH)
        def _(it):
            base = base0 + it
            pltpu.sync_copy(p0_hbm.at[pl.ds(base, CH)], i0_v)
            pltpu.sync_copy(p1_hbm.at[pl.ds(base, CH)], i1_v)
            pltpu.sync_copy(x_hbm.at[pl.ds(base, CH)], rows_v)
            pltpu.sync_copy(rows_v, out_hbm.at[i0_v])
            pltpu.sync_copy(rows_v, out_hbm.at[i1_v])

    return scatter_kernel(xt, pos0.reshape(-1), pos1.reshape(-1))


def _gemm_body(be_ref, nact_ref, xs_ref, wg_ref, bg_ref, wu_ref, bu_ref,
               wd_ref, bd_ref, out_ref):
    i = pl.program_id(0)
    j = pl.program_id(1)

    @pl.when(i < nact_ref[0])
    def _():
        @pl.when(j == 0)
        def _():
            out_ref[...] = jnp.zeros_like(out_ref)

        x = xs_ref[...].astype(jnp.bfloat16)
        hg = jnp.dot(x, wg_ref[0].astype(jnp.bfloat16),
                     preferred_element_type=jnp.float32)
        hg = hg + bg_ref[0]
        hu = jnp.dot(x, wu_ref[0].astype(jnp.bfloat16),
                     preferred_element_type=jnp.float32)
        hu = hu + bu_ref[0]
        h = (hg * jax.nn.sigmoid(hg)) * hu
        out_ref[...] += jnp.dot(h.astype(jnp.bfloat16),
                                wd_ref[0].astype(jnp.bfloat16),
                                preferred_element_type=jnp.float32)

        @pl.when(j == 0)
        def _():
            out_ref[...] += bd_ref[0]


def _grouped_gemm(xs, be, nact, W_gate, b_gate, W_up, b_up, W_down, b_down):
    d = D_MODEL

    def xmap(i, j, be_r, na_r):
        return (jnp.minimum(i, na_r[0] - 1), 0)

    def wmap_g(i, j, be_r, na_r):
        return (be_r[i], 0, jnp.where(i < na_r[0], j, 0))

    def bmap_g(i, j, be_r, na_r):
        return (be_r[i], 0, jnp.where(i < na_r[0], j, 0))

    def wmap_d(i, j, be_r, na_r):
        return (be_r[i], jnp.where(i < na_r[0], j, 0), 0)

    def bmap_d(i, j, be_r, na_r):
        return (be_r[i], 0, 0)

    grid_spec = pltpu.PrefetchScalarGridSpec(
        num_scalar_prefetch=2,
        grid=(NB, NF),
        in_specs=[
            pl.BlockSpec((M_BLK, d), xmap),
            pl.BlockSpec((1, d, FF_B), wmap_g),
            pl.BlockSpec((1, 1, FF_B), bmap_g),
            pl.BlockSpec((1, d, FF_B), wmap_g),
            pl.BlockSpec((1, 1, FF_B), bmap_g),
            pl.BlockSpec((1, FF_B, d), wmap_d),
            pl.BlockSpec((1, 1, d), bmap_d),
        ],
        out_specs=pl.BlockSpec((M_BLK, d), lambda i, j, be_r, na_r: (i, 0)),
    )
    return pl.pallas_call(
        _gemm_body,
        grid_spec=grid_spec,
        out_shape=jax.ShapeDtypeStruct((R_MAX, d), jnp.float32),
    )(be.reshape(NB), nact.reshape(1), xs, W_gate,
      b_gate.reshape(NUM_EXPERTS, 1, D_FF), W_up,
      b_up.reshape(NUM_EXPERTS, 1, D_FF), W_down,
      b_down.reshape(NUM_EXPERTS, 1, D_MODEL))


def _combine_gather_sc(o_sorted, pos0, pos1):
    mesh = plsc.VectorSubcoreMesh(core_axis_name="c", subcore_axis_name="s")
    per_w = T_TOKENS // SC_TILES

    @functools.partial(
        pl.kernel,
        out_type=[
            jax.ShapeDtypeStruct((T_TOKENS, D_MODEL), jnp.float32),
            jax.ShapeDtypeStruct((T_TOKENS, D_MODEL), jnp.float32),
        ],
        mesh=mesh,
        scratch_types=[
            pltpu.VMEM((CH,), jnp.int32),
            pltpu.VMEM((CH,), jnp.int32),
            pltpu.VMEM((CH, D_MODEL), jnp.float32),
            pltpu.VMEM((CH, D_MODEL), jnp.float32),
        ],
    )
    def gather_kernel(os_hbm, p0_hbm, p1_hbm, g0_hbm, g1_hbm,
                      i0_v, i1_v, r0_v, r1_v):
        wid = lax.axis_index("s") * SC_CORES + lax.axis_index("c")
        base0 = wid * per_w

        @pl.loop(0, per_w, step=CH)
        def _(it):
            base = base0 + it
            pltpu.sync_copy(p0_hbm.at[pl.ds(base, CH)], i0_v)
            pltpu.sync_copy(p1_hbm.at[pl.ds(base, CH)], i1_v)
            pltpu.sync_copy(os_hbm.at[i0_v], r0_v)
            pltpu.sync_copy(os_hbm.at[i1_v], r1_v)
            pltpu.sync_copy(r0_v, g0_hbm.at[pl.ds(base, CH)])
            pltpu.sync_copy(r1_v, g1_hbm.at[pl.ds(base, CH)])

    return gather_kernel(o_sorted, pos0.reshape(-1), pos1.reshape(-1))


def _combine_body(g0_ref, g1_ref, w0_ref, w1_ref, out_ref):
    out_ref[...] = w0_ref[...] * g0_ref[...] + w1_ref[...] * g1_ref[...]


def _combine(g0, g1, w0, w1):
    TB = 2048
    return pl.pallas_call(
        _combine_body,
        grid=(T_TOKENS // TB,),
        in_specs=[
            pl.BlockSpec((TB, D_MODEL), lambda t: (t, 0)),
            pl.BlockSpec((TB, D_MODEL), lambda t: (t, 0)),
            pl.BlockSpec((TB, 1), lambda t: (t, 0)),
            pl.BlockSpec((TB, 1), lambda t: (t, 0)),
        ],
        out_specs=pl.BlockSpec((TB, D_MODEL), lambda t: (t, 0)),
        out_shape=jax.ShapeDtypeStruct((T_TOKENS, D_MODEL), jnp.float32),
    )(g0, g1, w0, w1)


def kernel(x, W_router, b_router, W_gate, b_gate, W_up, b_up, W_down, b_down):
    b, s, d = x.shape
    xt = x.reshape(-1, d)
    w0, w1, pos0, pos1, be, nact = _router(xt, W_router, b_router)
    xs = _dispatch_sc(xt, pos0, pos1)
    o_sorted = _grouped_gemm(xs, be, nact, W_gate, b_gate, W_up, b_up,
                             W_down, b_down)
    g0, g1 = _combine_gather_sc(o_sorted, pos0, pos1)
    out = _combine(g0, g1, w0, w1)
    return out.reshape(b, s, d)
